# baseline (speedup 1.0000x reference)
import functools

import jax
import jax.numpy as jnp
from jax import lax
from jax.experimental import pallas as pl
from jax.experimental.pallas import tpu as pltpu
from jax.experimental.pallas import tpu_sc as plsc

_K = 16
_ROWS_KNN = 1024
_ROWS_PROJ = 512
_ROWS_ATTN = 1024
_GATHER_CHUNK = 256


def _knn_body(n, xyzc_ref, xyzq_ref, idx_ref):
    b = pl.program_id(0)
    xc = xyzc_ref[0]
    xq = xyzq_ref[0]
    dx = xc[:, 0:1] - xq[0:1, :]
    d = dx * dx
    dy = xc[:, 1:2] - xq[1:2, :]
    d = d + dy * dy
    dz = xc[:, 2:3] - xq[2:3, :]
    d = d + dz * dz
    keys = lax.bitcast_convert_type(d, jnp.int32)
    keys = (keys & jnp.int32(-2048)) | lax.broadcasted_iota(jnp.int32, d.shape, 0)
    kf = lax.bitcast_convert_type(keys + jnp.int32(0x00800000), jnp.float32)
    base = b * n
    m = jnp.min(kf, axis=0)
    idx_ref[0, 0, :] = (lax.bitcast_convert_type(m, jnp.int32) & 2047) + base
    for j in range(1, _K):
        m = jnp.min(jnp.where(kf > m[None, :], kf, jnp.float32(jnp.inf)), axis=0)
        idx_ref[0, j, :] = (lax.bitcast_convert_type(m, jnp.int32) & 2047) + base


def _knn(xyzp, xyzt):
    b, n, _ = xyzp.shape
    rt = _ROWS_KNN
    return pl.pallas_call(
        functools.partial(_knn_body, n),
        grid=(b, n // rt),
        in_specs=[
            pl.BlockSpec((1, n, 8), lambda i, t: (i, 0, 0)),
            pl.BlockSpec((1, 8, rt), lambda i, t: (i, 0, t)),
        ],
        out_specs=pl.BlockSpec((1, _K, rt), lambda i, t: (i, 0, t)),
        out_shape=jax.ShapeDtypeStruct((b, _K, n), jnp.int32),
    )(xyzp, xyzt)


def _proj_body(f_ref, w1t_ref, b1_ref, wqt_ref, wkt_ref, wvt_ref, q_ref, kv_ref):
    x = jnp.dot(f_ref[...], w1t_ref[...], preferred_element_type=jnp.float32)
    x = x + b1_ref[...]
    q_ref[...] = jnp.dot(x, wqt_ref[...], preferred_element_type=jnp.float32)
    kk = jnp.dot(x, wkt_ref[...], preferred_element_type=jnp.float32)
    vv = jnp.dot(x, wvt_ref[...], preferred_element_type=jnp.float32)
    kv_ref[...] = jnp.concatenate([kk, vv], axis=1)


def _proj(f2, w1t, b1, wqt, wkt, wvt):
    bn, dp = f2.shape
    dm = w1t.shape[1]
    rb = _ROWS_PROJ
    return pl.pallas_call(
        _proj_body,
        grid=(bn // rb,),
        in_specs=[
            pl.BlockSpec((rb, dp), lambda i: (i, 0)),
            pl.BlockSpec((dp, dm), lambda i: (0, 0)),
            pl.BlockSpec((1, dm), lambda i: (0, 0)),
            pl.BlockSpec((dm, dm), lambda i: (0, 0)),
            pl.BlockSpec((dm, dm), lambda i: (0, 0)),
            pl.BlockSpec((dm, dm), lambda i: (0, 0)),
        ],
        out_specs=[
            pl.BlockSpec((rb, dm), lambda i: (i, 0)),
            pl.BlockSpec((rb, 2 * dm), lambda i: (i, 0)),
        ],
        out_shape=[
            jax.ShapeDtypeStruct((bn, dm), jnp.float32),
            jax.ShapeDtypeStruct((bn, 2 * dm), jnp.float32),
        ],
    )(f2, w1t, b1, wqt, wkt, wvt)


def _sc_gather(kv, xs, ys, zs, idx3):
    info = plsc.get_sparse_core_info()
    nc, ns = info.num_cores, info.num_subcores
    nw = nc * ns
    b, k, n = idx3.shape
    bn = xs.shape[0]
    r = k * bn
    per_w = r // nw
    ch = _GATHER_CHUNK
    nch = per_w // ch
    ng = ch // 16
    pts_w = bn // nw
    wkv = kv.shape[1]
    mesh = plsc.VectorSubcoreMesh(core_axis_name="c", subcore_axis_name="s")

    @functools.partial(
        pl.kernel,
        out_type=[
            jax.ShapeDtypeStruct((r, wkv), jnp.float32),
            jax.ShapeDtypeStruct((bn * k,), jnp.float32),
            jax.ShapeDtypeStruct((bn * k,), jnp.float32),
            jax.ShapeDtypeStruct((bn * k,), jnp.float32),
        ],
        mesh=mesh,
        scratch_types=[
            pltpu.VMEM((bn,), jnp.float32),
            pltpu.VMEM((bn,), jnp.float32),
            pltpu.VMEM((bn,), jnp.float32),
            pltpu.VMEM((per_w,), jnp.int32),
            pltpu.VMEM((ch, wkv), jnp.float32),
            pltpu.VMEM((ch, wkv), jnp.float32),
            pltpu.VMEM((k, pts_w), jnp.int32),
            pltpu.VMEM((pts_w * k,), jnp.float32),
            pltpu.VMEM((pts_w * k,), jnp.float32),
            pltpu.VMEM((pts_w * k,), jnp.float32),
            pltpu.SemaphoreType.DMA,
            pltpu.SemaphoreType.DMA,
        ],
        compiler_params=pltpu.CompilerParams(needs_layout_passes=False),
    )
    def gather_k(kv_hbm, xs_hbm, ys_hbm, zs_hbm, idx_hbm,
                 kvg_hbm, dx_hbm, dy_hbm, dz_hbm,
                 xt, yt, zt, idx_all, kvb0, kvb1, idxm, dbx, dby, dbz, s0, s1):
        wid = lax.axis_index("s") * nc + lax.axis_index("c")
        pltpu.sync_copy(xs_hbm, xt)
        pltpu.sync_copy(ys_hbm, yt)
        pltpu.sync_copy(zs_hbm, zt)
        lanes = lax.iota(jnp.int32, 16)

        jw = wid // 2
        ibase = (wid % 2) * per_w
        for h in range(per_w // n):
            pltpu.sync_copy(idx_hbm.at[ibase // n + h, jw, :],
                            idx_all.at[pl.ds(h * n, n)])

        def start(c, kb, sem):
            return pltpu.async_copy(
                kv_hbm.at[idx_all.at[pl.ds(c * ch, ch)]], kb, sem)

        def drain(c, kb, cp):
            cp.wait()
            pltpu.sync_copy(kb, kvg_hbm.at[pl.ds(jw * bn + ibase + c * ch, ch)])

        inflight = [
            (0, kvb0, s0, start(0, kvb0, s0)),
            (1, kvb1, s1, start(1, kvb1, s1)),
        ]

        i0 = wid * pts_w
        b2 = i0 // n
        n0 = i0 % n
        pltpu.sync_copy(idx_hbm.at[b2, :, pl.ds(n0, pts_w)], idxm)

        def drow(j, carry):
            for t in range(pts_w // 16):
                jv = idxm[j, pl.ds(t * 16, 16)]
                iv = i0 + t * 16 + lanes
                fidx = t * 256 + lanes * k + j
                dd = plsc.load_gather(xt, [iv]) - plsc.load_gather(xt, [jv])
                plsc.store_scatter(dbx, [fidx], dd)
                dd = plsc.load_gather(yt, [iv]) - plsc.load_gather(yt, [jv])
                plsc.store_scatter(dby, [fidx], dd)
                dd = plsc.load_gather(zt, [iv]) - plsc.load_gather(zt, [jv])
                plsc.store_scatter(dbz, [fidx], dd)
            return carry

        lax.fori_loop(0, k, drow, 0)
        pltpu.sync_copy(dbx, dx_hbm.at[pl.ds(i0 * k, pts_w * k)])
        pltpu.sync_copy(dby, dy_hbm.at[pl.ds(i0 * k, pts_w * k)])
        pltpu.sync_copy(dbz, dz_hbm.at[pl.ds(i0 * k, pts_w * k)])

        for c in range(2, nch + 2):
            cd, kb, sem, cp = inflight.pop(0)
            drain(cd, kb, cp)
            if c < nch:
                inflight.append((c, kb, sem, start(c, kb, sem)))

    return gather_k(kv, xs, ys, zs, idx3)


def _attn_body(kvg_ref, dx_ref, dy_ref, dz_ref, q_ref, f_ref, wd1x_ref, bd1_ref,
               wd2t_ref, bd2_ref, wg1t_ref, bg1_ref, wg2t_ref, bg2_ref,
               w2t_ref, b2_ref, o_ref):
    q = q_ref[...]
    d48 = jnp.concatenate([dx_ref[...], dy_ref[...], dz_ref[...]], axis=1)
    h_all = jnp.dot(d48, wd1x_ref[...], preferred_element_type=jnp.float32)
    bd1 = bd1_ref[...]
    wd2t = wd2t_ref[...]
    bd2 = bd2_ref[...]
    wg1t = wg1t_ref[...]
    bg1 = bg1_ref[...]
    wg2t = wg2t_ref[...]
    bg2 = bg2_ref[...]
    dm = q.shape[1]
    a_list = []
    u_list = []
    for j in range(_K):
        kvj = kvg_ref[j]
        kj = kvj[:, :dm]
        vj = kvj[:, dm:]
        h = jnp.maximum(h_all[:, j * dm:(j + 1) * dm] + bd1, 0.0)
        pos = jnp.dot(h, wd2t, preferred_element_type=jnp.float32) + bd2
        g = q - kj + pos
        h2 = jnp.maximum(jnp.dot(g, wg1t, preferred_element_type=jnp.float32) + bg1, 0.0)
        a_list.append(jnp.dot(h2, wg2t, preferred_element_type=jnp.float32) + bg2)
        u_list.append(vj + pos)
    m = a_list[0]
    for j in range(1, _K):
        m = jnp.maximum(m, a_list[j])
    es = [jnp.exp(a - m) for a in a_list]
    s = es[0]
    for j in range(1, _K):
        s = s + es[j]
    inv = 1.0 / s
    attn = [e * inv for e in es]
    t = attn[0]
    for j in range(1, _K):
        t = t + attn[j]
    invt = 1.0 / jnp.maximum(t, 1e-12)
    r = attn[0] * u_list[0]
    for j in range(1, _K):
        r = r + attn[j] * u_list[j]
    r = r * invt
    out = jnp.dot(r, w2t_ref[...], preferred_element_type=jnp.float32)
    o_ref[...] = out + b2_ref[...] + f_ref[...]


def _attn(kvg, dx, dy, dz, q, f2, wd1x, bd1, wd2t, bd2, wg1t, bg1, wg2t, bg2, w2t, b2):
    bn, dp = f2.shape
    dm = q.shape[1]
    rt = _ROWS_ATTN
    return pl.pallas_call(
        _attn_body,
        grid=(bn // rt,),
        in_specs=[
            pl.BlockSpec((_K, rt, 2 * dm), lambda i: (0, i, 0)),
            pl.BlockSpec((rt, _K), lambda i: (i, 0)),
            pl.BlockSpec((rt, _K), lambda i: (i, 0)),
            pl.BlockSpec((rt, _K), lambda i: (i, 0)),
            pl.BlockSpec((rt, dm), lambda i: (i, 0)),
            pl.BlockSpec((rt, dp), lambda i: (i, 0)),
            pl.BlockSpec((3 * _K, _K * dm), lambda i: (0, 0)),
            pl.BlockSpec((1, dm), lambda i: (0, 0)),
            pl.BlockSpec((dm, dm), lambda i: (0, 0)),
            pl.BlockSpec((1, dm), lambda i: (0, 0)),
            pl.BlockSpec((dm, dm), lambda i: (0, 0)),
            pl.BlockSpec((1, dm), lambda i: (0, 0)),
            pl.BlockSpec((dm, dm), lambda i: (0, 0)),
            pl.BlockSpec((1, dm), lambda i: (0, 0)),
            pl.BlockSpec((dm, dp), lambda i: (0, 0)),
            pl.BlockSpec((1, dp), lambda i: (0, 0)),
        ],
        out_specs=pl.BlockSpec((rt, dp), lambda i: (i, 0)),
        out_shape=jax.ShapeDtypeStruct((bn, dp), jnp.float32),
    )(kvg, dx, dy, dz, q, f2, wd1x, bd1, wd2t, bd2, wg1t, bg1, wg2t, bg2, w2t, b2)


def kernel(xyz, features, W1, b1, W2, b2, Wd1, bd1, Wd2, bd2, Wg1, bg1, Wg2, bg2, Wq, Wk, Wv):
    b, n, dp = features.shape
    dm = W1.shape[0]
    bn = b * n
    xyzp = jnp.pad(xyz, ((0, 0), (0, 0), (0, 5)))
    xyzt = jnp.transpose(xyzp, (0, 2, 1))
    idx = _knn(xyzp, xyzt)
    f2 = features.reshape(bn, dp)
    q, kv = _proj(f2, W1.T, b1.reshape(1, -1), Wq.T, Wk.T, Wv.T)
    x3 = xyz.reshape(bn, 3)
    kvg, dx, dy, dz = _sc_gather(kv, x3[:, 0], x3[:, 1], x3[:, 2], idx)
    wd1x = (Wd1.T[:, None, None, :] * jnp.eye(_K, dtype=Wd1.dtype)[None, :, :, None])
    wd1x = wd1x.reshape(3 * _K, _K * dm)
    out = _attn(
        kvg.reshape(_K, bn, 2 * dm),
        dx.reshape(bn, _K), dy.reshape(bn, _K), dz.reshape(bn, _K), q, f2,
        wd1x, bd1.reshape(1, -1), Wd2.T, bd2.reshape(1, -1),
        Wg1.T, bg1.reshape(1, -1), Wg2.T, bg2.reshape(1, -1), W2.T, b2.reshape(1, -1))
    return out.reshape(b, n, dp)


# speedup vs baseline: 1.0190x; 1.0190x over previous
import functools

import jax
import jax.numpy as jnp
from jax import lax
from jax.experimental import pallas as pl
from jax.experimental.pallas import tpu as pltpu
from jax.experimental.pallas import tpu_sc as plsc

_K = 16
_ROWS_KNN = 512
_ROWS_PROJ = 512
_ROWS_ATTN = 1024
_GATHER_CHUNK = 256


def _knn_body(n, xyzc_ref, xyzq_ref, idx_ref):
    b = pl.program_id(0)
    xc = xyzc_ref[0]
    xq = xyzq_ref[0]
    dx = xc[:, 0:1] - xq[0:1, :]
    d = dx * dx
    dy = xc[:, 1:2] - xq[1:2, :]
    d = d + dy * dy
    dz = xc[:, 2:3] - xq[2:3, :]
    d = d + dz * dz
    keys = lax.bitcast_convert_type(d, jnp.int32)
    keys = (keys & jnp.int32(-2048)) | lax.broadcasted_iota(jnp.int32, d.shape, 0)
    kf = lax.bitcast_convert_type(keys + jnp.int32(0x00800000), jnp.float32)
    base = b * n
    m = jnp.min(kf, axis=0)
    idx_ref[0, 0, :] = (lax.bitcast_convert_type(m, jnp.int32) & 2047) + base
    for j in range(1, _K):
        m = jnp.min(jnp.where(kf > m[None, :], kf, jnp.float32(jnp.inf)), axis=0)
        idx_ref[0, j, :] = (lax.bitcast_convert_type(m, jnp.int32) & 2047) + base


def _knn(xyzp, xyzt):
    b, n, _ = xyzp.shape
    rt = _ROWS_KNN
    return pl.pallas_call(
        functools.partial(_knn_body, n),
        grid=(b, n // rt),
        in_specs=[
            pl.BlockSpec((1, n, 8), lambda i, t: (i, 0, 0)),
            pl.BlockSpec((1, 8, rt), lambda i, t: (i, 0, t)),
        ],
        out_specs=pl.BlockSpec((1, _K, rt), lambda i, t: (i, 0, t)),
        out_shape=jax.ShapeDtypeStruct((b, _K, n), jnp.int32),
    )(xyzp, xyzt)


def _proj_body(f_ref, w1t_ref, b1_ref, wqt_ref, wkt_ref, wvt_ref, q_ref, kv_ref):
    x = jnp.dot(f_ref[...], w1t_ref[...], preferred_element_type=jnp.float32)
    x = x + b1_ref[...]
    q_ref[...] = jnp.dot(x, wqt_ref[...], preferred_element_type=jnp.float32)
    kk = jnp.dot(x, wkt_ref[...], preferred_element_type=jnp.float32)
    vv = jnp.dot(x, wvt_ref[...], preferred_element_type=jnp.float32)
    kv_ref[...] = jnp.concatenate([kk, vv], axis=1)


def _proj(f2, w1t, b1, wqt, wkt, wvt):
    bn, dp = f2.shape
    dm = w1t.shape[1]
    rb = _ROWS_PROJ
    return pl.pallas_call(
        _proj_body,
        grid=(bn // rb,),
        in_specs=[
            pl.BlockSpec((rb, dp), lambda i: (i, 0)),
            pl.BlockSpec((dp, dm), lambda i: (0, 0)),
            pl.BlockSpec((1, dm), lambda i: (0, 0)),
            pl.BlockSpec((dm, dm), lambda i: (0, 0)),
            pl.BlockSpec((dm, dm), lambda i: (0, 0)),
            pl.BlockSpec((dm, dm), lambda i: (0, 0)),
        ],
        out_specs=[
            pl.BlockSpec((rb, dm), lambda i: (i, 0)),
            pl.BlockSpec((rb, 2 * dm), lambda i: (i, 0)),
        ],
        out_shape=[
            jax.ShapeDtypeStruct((bn, dm), jnp.float32),
            jax.ShapeDtypeStruct((bn, 2 * dm), jnp.float32),
        ],
    )(f2, w1t, b1, wqt, wkt, wvt)


def _sc_gather(kv, xs, ys, zs, idx3):
    info = plsc.get_sparse_core_info()
    nc, ns = info.num_cores, info.num_subcores
    nw = nc * ns
    b, k, n = idx3.shape
    bn = xs.shape[0]
    r = k * bn
    per_w = r // nw
    ch = _GATHER_CHUNK
    nch = per_w // ch
    ng = ch // 16
    pts_w = bn // nw
    wkv = kv.shape[1]
    mesh = plsc.VectorSubcoreMesh(core_axis_name="c", subcore_axis_name="s")

    @functools.partial(
        pl.kernel,
        out_type=[
            jax.ShapeDtypeStruct((r, wkv), jnp.float32),
            jax.ShapeDtypeStruct((bn * k,), jnp.float32),
            jax.ShapeDtypeStruct((bn * k,), jnp.float32),
            jax.ShapeDtypeStruct((bn * k,), jnp.float32),
        ],
        mesh=mesh,
        scratch_types=[
            pltpu.VMEM((bn,), jnp.float32),
            pltpu.VMEM((bn,), jnp.float32),
            pltpu.VMEM((bn,), jnp.float32),
            pltpu.VMEM((per_w,), jnp.int32),
            pltpu.VMEM((ch, wkv), jnp.float32),
            pltpu.VMEM((ch, wkv), jnp.float32),
            pltpu.VMEM((k, pts_w), jnp.int32),
            pltpu.VMEM((pts_w * k,), jnp.float32),
            pltpu.VMEM((pts_w * k,), jnp.float32),
            pltpu.VMEM((pts_w * k,), jnp.float32),
            pltpu.SemaphoreType.DMA,
            pltpu.SemaphoreType.DMA,
        ],
        compiler_params=pltpu.CompilerParams(needs_layout_passes=False),
    )
    def gather_k(kv_hbm, xs_hbm, ys_hbm, zs_hbm, idx_hbm,
                 kvg_hbm, dx_hbm, dy_hbm, dz_hbm,
                 xt, yt, zt, idx_all, kvb0, kvb1, idxm, dbx, dby, dbz, s0, s1):
        wid = lax.axis_index("s") * nc + lax.axis_index("c")
        pltpu.sync_copy(xs_hbm, xt)
        pltpu.sync_copy(ys_hbm, yt)
        pltpu.sync_copy(zs_hbm, zt)
        lanes = lax.iota(jnp.int32, 16)

        jw = wid // 2
        ibase = (wid % 2) * per_w
        for h in range(per_w // n):
            pltpu.sync_copy(idx_hbm.at[ibase // n + h, jw, :],
                            idx_all.at[pl.ds(h * n, n)])

        def start(c, kb, sem):
            return pltpu.async_copy(
                kv_hbm.at[idx_all.at[pl.ds(c * ch, ch)]], kb, sem)

        def drain(c, kb, cp):
            cp.wait()
            pltpu.sync_copy(kb, kvg_hbm.at[pl.ds(jw * bn + ibase + c * ch, ch)])

        inflight = [
            (0, kvb0, s0, start(0, kvb0, s0)),
            (1, kvb1, s1, start(1, kvb1, s1)),
        ]

        i0 = wid * pts_w
        b2 = i0 // n
        n0 = i0 % n
        pltpu.sync_copy(idx_hbm.at[b2, :, pl.ds(n0, pts_w)], idxm)

        def drow(j, carry):
            for t in range(pts_w // 16):
                jv = idxm[j, pl.ds(t * 16, 16)]
                iv = i0 + t * 16 + lanes
                fidx = t * 256 + lanes * k + j
                dd = plsc.load_gather(xt, [iv]) - plsc.load_gather(xt, [jv])
                plsc.store_scatter(dbx, [fidx], dd)
                dd = plsc.load_gather(yt, [iv]) - plsc.load_gather(yt, [jv])
                plsc.store_scatter(dby, [fidx], dd)
                dd = plsc.load_gather(zt, [iv]) - plsc.load_gather(zt, [jv])
                plsc.store_scatter(dbz, [fidx], dd)
            return carry

        lax.fori_loop(0, k, drow, 0)
        pltpu.sync_copy(dbx, dx_hbm.at[pl.ds(i0 * k, pts_w * k)])
        pltpu.sync_copy(dby, dy_hbm.at[pl.ds(i0 * k, pts_w * k)])
        pltpu.sync_copy(dbz, dz_hbm.at[pl.ds(i0 * k, pts_w * k)])

        for c in range(2, nch + 2):
            cd, kb, sem, cp = inflight.pop(0)
            drain(cd, kb, cp)
            if c < nch:
                inflight.append((c, kb, sem, start(c, kb, sem)))

    return gather_k(kv, xs, ys, zs, idx3)


def _attn_body(kvg_ref, dx_ref, dy_ref, dz_ref, q_ref, f_ref, wd1x_ref, bd1_ref,
               wd2t_ref, bd2_ref, wg1t_ref, bg1_ref, wg2t_ref, bg2_ref,
               w2t_ref, b2_ref, o_ref):
    q = q_ref[...]
    d48 = jnp.concatenate([dx_ref[...], dy_ref[...], dz_ref[...]], axis=1)
    h_all = jnp.dot(d48, wd1x_ref[...], preferred_element_type=jnp.float32)
    bd1 = bd1_ref[...]
    wd2t = wd2t_ref[...]
    bd2 = bd2_ref[...]
    wg1t = wg1t_ref[...]
    bg1 = bg1_ref[...]
    wg2t = wg2t_ref[...]
    bg2 = bg2_ref[...]
    dm = q.shape[1]
    a_list = []
    u_list = []
    for j in range(_K):
        kvj = kvg_ref[j]
        kj = kvj[:, :dm]
        vj = kvj[:, dm:]
        h = jnp.maximum(h_all[:, j * dm:(j + 1) * dm] + bd1, 0.0)
        pos = jnp.dot(h, wd2t, preferred_element_type=jnp.float32) + bd2
        g = q - kj + pos
        h2 = jnp.maximum(jnp.dot(g, wg1t, preferred_element_type=jnp.float32) + bg1, 0.0)
        a_list.append(jnp.dot(h2, wg2t, preferred_element_type=jnp.float32) + bg2)
        u_list.append(vj + pos)
    m = a_list[0]
    for j in range(1, _K):
        m = jnp.maximum(m, a_list[j])
    es = [jnp.exp(a - m) for a in a_list]
    s = es[0]
    for j in range(1, _K):
        s = s + es[j]
    inv = 1.0 / s
    attn = [e * inv for e in es]
    t = attn[0]
    for j in range(1, _K):
        t = t + attn[j]
    invt = 1.0 / jnp.maximum(t, 1e-12)
    r = attn[0] * u_list[0]
    for j in range(1, _K):
        r = r + attn[j] * u_list[j]
    r = r * invt
    out = jnp.dot(r, w2t_ref[...], preferred_element_type=jnp.float32)
    o_ref[...] = out + b2_ref[...] + f_ref[...]


def _attn(kvg, dx, dy, dz, q, f2, wd1x, bd1, wd2t, bd2, wg1t, bg1, wg2t, bg2, w2t, b2):
    bn, dp = f2.shape
    dm = q.shape[1]
    rt = _ROWS_ATTN
    return pl.pallas_call(
        _attn_body,
        grid=(bn // rt,),
        in_specs=[
            pl.BlockSpec((_K, rt, 2 * dm), lambda i: (0, i, 0)),
            pl.BlockSpec((rt, _K), lambda i: (i, 0)),
            pl.BlockSpec((rt, _K), lambda i: (i, 0)),
            pl.BlockSpec((rt, _K), lambda i: (i, 0)),
            pl.BlockSpec((rt, dm), lambda i: (i, 0)),
            pl.BlockSpec((rt, dp), lambda i: (i, 0)),
            pl.BlockSpec((3 * _K, _K * dm), lambda i: (0, 0)),
            pl.BlockSpec((1, dm), lambda i: (0, 0)),
            pl.BlockSpec((dm, dm), lambda i: (0, 0)),
            pl.BlockSpec((1, dm), lambda i: (0, 0)),
            pl.BlockSpec((dm, dm), lambda i: (0, 0)),
            pl.BlockSpec((1, dm), lambda i: (0, 0)),
            pl.BlockSpec((dm, dm), lambda i: (0, 0)),
            pl.BlockSpec((1, dm), lambda i: (0, 0)),
            pl.BlockSpec((dm, dp), lambda i: (0, 0)),
            pl.BlockSpec((1, dp), lambda i: (0, 0)),
        ],
        out_specs=pl.BlockSpec((rt, dp), lambda i: (i, 0)),
        out_shape=jax.ShapeDtypeStruct((bn, dp), jnp.float32),
    )(kvg, dx, dy, dz, q, f2, wd1x, bd1, wd2t, bd2, wg1t, bg1, wg2t, bg2, w2t, b2)


def kernel(xyz, features, W1, b1, W2, b2, Wd1, bd1, Wd2, bd2, Wg1, bg1, Wg2, bg2, Wq, Wk, Wv):
    b, n, dp = features.shape
    dm = W1.shape[0]
    bn = b * n
    xyzp = jnp.pad(xyz, ((0, 0), (0, 0), (0, 5)))
    xyzt = jnp.transpose(xyzp, (0, 2, 1))
    idx = _knn(xyzp, xyzt)
    f2 = features.reshape(bn, dp)
    q, kv = _proj(f2, W1.T, b1.reshape(1, -1), Wq.T, Wk.T, Wv.T)
    x3 = xyz.reshape(bn, 3)
    kvg, dx, dy, dz = _sc_gather(kv, x3[:, 0], x3[:, 1], x3[:, 2], idx)
    wd1x = (Wd1.T[:, None, None, :] * jnp.eye(_K, dtype=Wd1.dtype)[None, :, :, None])
    wd1x = wd1x.reshape(3 * _K, _K * dm)
    out = _attn(
        kvg.reshape(_K, bn, 2 * dm),
        dx.reshape(bn, _K), dy.reshape(bn, _K), dz.reshape(bn, _K), q, f2,
        wd1x, bd1.reshape(1, -1), Wd2.T, bd2.reshape(1, -1),
        Wg1.T, bg1.reshape(1, -1), Wg2.T, bg2.reshape(1, -1), W2.T, b2.reshape(1, -1))
    return out.reshape(b, n, dp)
